# baseline (speedup 1.0000x reference)
import jax
import jax.numpy as jnp
from jax import lax
from jax.experimental import pallas as pl
from jax.experimental.pallas import tpu as pltpu

_RADIUS = 128
_D = 64
_T = 2048
_E_PAD = 264
_BR = 16
_NSEM = 4
_MROWS = _T


def _expand_kernel(e_ref, out_ref, ccf_ref, sems):
    i = pl.program_id(0)

    @pl.when(i == 0)
    def _build_strip():
        e = e_ref[...]
        m_iota = lax.broadcasted_iota(jnp.int32, (_MROWS, _E_PAD), 0)
        b_iota = lax.broadcasted_iota(jnp.int32, (_MROWS, _E_PAD), 1)
        for r in range(_BR):
            for lhi in range(2):
                sel = jnp.clip(2176 - 2 * m_iota - lhi + r, 0, 2 * _RADIUS)
                p = (b_iota == sel).astype(jnp.float32)
                ccf_ref[r, :, 64 * lhi:64 * lhi + 64] = jnp.dot(
                    p, e, preferred_element_type=jnp.float32,
                    precision=lax.Precision.HIGHEST)

    w = 1024 - 8 * i
    slot = lax.rem(i, _NSEM)

    @pl.when(i >= _NSEM)
    def _drain_prev():
        pltpu.make_async_copy(
            ccf_ref.at[:, pl.ds(0, 1024), :], out_ref.at[pl.ds(0, _BR)],
            sems.at[slot]).wait()

    pltpu.make_async_copy(
        ccf_ref.at[:, pl.ds(w, 1024), :], out_ref.at[pl.ds(_BR * i, _BR)],
        sems.at[slot]).start()

    @pl.when(i == _T // _BR - 1)
    def _drain_all():
        for s in range(_NSEM):
            pltpu.make_async_copy(
                ccf_ref.at[:, pl.ds(0, 1024), :], out_ref.at[pl.ds(0, _BR)],
                sems.at[s]).wait()


def kernel(num_frames, embed):
    del num_frames
    e = jnp.pad(embed, ((0, _E_PAD - 2 * _RADIUS - 1), (0, 0)))
    out = pl.pallas_call(
        _expand_kernel,
        grid=(_T // _BR,),
        in_specs=[pl.BlockSpec((_E_PAD, _D), lambda i: (0, 0))],
        out_specs=pl.BlockSpec(memory_space=pltpu.MemorySpace.HBM),
        out_shape=jax.ShapeDtypeStruct((_T, _T * _D // 128, 128), jnp.float32),
        scratch_shapes=[
            pltpu.VMEM((_BR, _MROWS, 128), jnp.float32),
            pltpu.SemaphoreType.DMA((_NSEM,)),
        ],
    )(e)
    return out.reshape(_T, _T, _D)
